# Initial kernel scaffold; baseline (speedup 1.0000x reference)
import jax
import jax.numpy as jnp
from jax.experimental import pallas as pl


def kernel(h, edge_index, e, W_att, W_edge, W_e2n):
    raise NotImplementedError("write your pallas kernel here")


# speedup vs baseline: 18.9821x; 18.9821x over previous
import functools

import jax
import jax.numpy as jnp
from jax import lax
from jax.experimental import pallas as pl
from jax.experimental.pallas import tpu as pltpu
from jax.experimental.pallas import tpu_sc as plsc

N = 10000
E = 320000
D = 128
ED = 16

NC = 2
NS = 16
NW = NC * NS
L = 16

EPW = E // NW
BLK = 400
NBLK = EPW // BLK
RPS = 100
NSTR = BLK // RPS
GRP = BLK // L

NPAD = 10240
STRIPE = NPAD // NS


def _node_alpha_body(h_ref, wa_ref, o1_ref, o2_ref):
    hb = h_ref[...]
    w1 = wa_ref[0, :D]
    w2 = wa_ref[0, D:2 * D]
    o1_ref[...] = jnp.dot(hb, w1, preferred_element_type=jnp.float32)
    o2_ref[...] = jnp.dot(hb, w2, preferred_element_type=jnp.float32)


def _node_alpha(h, W_att):
    return pl.pallas_call(
        _node_alpha_body,
        out_shape=[
            jax.ShapeDtypeStruct((N,), jnp.float32),
            jax.ShapeDtypeStruct((N,), jnp.float32),
        ],
    )(h, W_att)


def _edge_prep_body(e_ref, w1_ref, w2_ref, ex_ref, b_ref):
    eb = e_ref[...]
    ex = jnp.dot(eb, w1_ref[...], preferred_element_type=jnp.float32)
    ex_ref[...] = ex
    b_ref[...] = jnp.dot(ex, w2_ref[...], preferred_element_type=jnp.float32)


def _edge_prep(e8, W1, W2):
    rows = E // 8
    blk = 2000
    grid = (rows // blk,)
    return pl.pallas_call(
        _edge_prep_body,
        grid=grid,
        in_specs=[
            pl.BlockSpec((blk, 8 * ED), lambda i: (i, 0)),
            pl.BlockSpec((8 * ED, 8 * ED), lambda i: (0, 0)),
            pl.BlockSpec((8 * ED, 8), lambda i: (0, 0)),
        ],
        out_specs=[
            pl.BlockSpec((blk, 8 * ED), lambda i: (i, 0)),
            pl.BlockSpec((blk, 8), lambda i: (i, 0)),
        ],
        out_shape=[
            jax.ShapeDtypeStruct((rows, 8 * ED), jnp.float32),
            jax.ShapeDtypeStruct((rows, 8), jnp.float32),
        ],
    )(e8, W1, W2)


def _sc_body(src_hbm, dst_hbm, asrc_hbm, adst_hbm, beta_hbm, ex_hbm,
             s_out, den_out,
             asrc_v, adst_v, den_v, src_v, dstf_v, beta_v, ex_v,
             row16_v, zbuf_v, dsum_v, s_sh, den_sh):
    cid = lax.axis_index("c")
    sid = lax.axis_index("s")
    wid = cid * NS + sid
    ebase = wid * EPW

    zero16 = jnp.zeros((L,), jnp.float32)

    pltpu.sync_copy(asrc_hbm, asrc_v)
    pltpu.sync_copy(adst_hbm, adst_v)

    def _z_rows(i, _):
        zbuf_v[i, :] = zero16
        return _
    lax.fori_loop(0, STRIPE, _z_rows, None)

    def _z_den(i, _):
        den_v[pl.ds(i * L, L)] = zero16
        return _
    lax.fori_loop(0, NPAD // L, _z_den, None)

    pltpu.sync_copy(zbuf_v, s_sh.at[pl.ds(sid * STRIPE, STRIPE)])
    plsc.subcore_barrier()

    def _block(blk, _):
        base = ebase + blk * BLK
        pltpu.sync_copy(src_hbm.at[pl.ds(base, BLK)], src_v)
        pltpu.sync_copy(dst_hbm.at[pl.ds(base, BLK)], dstf_v)
        pltpu.sync_copy(beta_hbm.at[pl.ds(base, BLK)], beta_v)
        pltpu.sync_copy(ex_hbm.at[pl.ds(base, BLK), :], ex_v)

        def _grp(g, _):
            o = g * L
            sv = src_v[pl.ds(o, L)]
            dv = dstf_v[pl.ds(o, L)]
            a1 = plsc.load_gather(asrc_v, [sv])
            a2 = plsc.load_gather(adst_v, [dv])
            att = a1 + a2 + beta_v[pl.ds(o, L)]
            att = jnp.maximum(att, att * 0.01)
            ee = jnp.exp(att)
            plsc.addupdate_scatter(den_v, [dv], ee)
            for j in range(L):
                row16_v[j, :] = ex_v[o + j, :] * ee[j]
            pltpu.sync_copy(row16_v, s_sh.at[dv], add=True)
            return _
        lax.fori_loop(0, GRP, _grp, None)
        return _

    lax.fori_loop(0, NBLK, _block, None)

    pltpu.sync_copy(den_v, den_sh.at[sid])
    plsc.subcore_barrier()

    sbase = sid * STRIPE
    for t in range(NS):
        pltpu.sync_copy(den_sh.at[t, pl.ds(sbase, STRIPE)], dsum_v.at[t])

    def _red(g, _):
        sl = pl.ds(g * L, L)
        acc = dsum_v[0, sl]
        for t in range(1, NS):
            acc = acc + dsum_v[t, sl]
        dsum_v[0, sl] = acc
        return _
    lax.fori_loop(0, STRIPE // L, _red, None)

    pltpu.sync_copy(dsum_v.at[0], den_out.at[cid, pl.ds(sbase, STRIPE)])
    pltpu.sync_copy(s_sh.at[pl.ds(sbase, STRIPE)],
                    s_out.at[cid, pl.ds(sbase, STRIPE), :])


def _sc_agg(src, dst, a_src, a_dst, beta, ex):
    mesh = plsc.VectorSubcoreMesh(core_axis_name="c", subcore_axis_name="s",
                                  num_cores=NC, num_subcores=NS)
    f32 = jnp.float32
    kern = pl.kernel(
        _sc_body,
        out_type=[
            jax.ShapeDtypeStruct((NC, NPAD, ED), f32),
            jax.ShapeDtypeStruct((NC, NPAD), f32),
        ],
        mesh=mesh,
        compiler_params=pltpu.CompilerParams(needs_layout_passes=False,
                                             use_tc_tiling_on_sc=False),
        scratch_types=[
            pltpu.VMEM((N,), f32),
            pltpu.VMEM((N,), f32),
            pltpu.VMEM((NPAD,), f32),
            pltpu.VMEM((BLK,), jnp.int32),
            pltpu.VMEM((BLK,), jnp.int32),
            pltpu.VMEM((BLK,), f32),
            pltpu.VMEM((BLK, ED), f32),
            pltpu.VMEM((L, ED), f32),
            pltpu.VMEM((STRIPE, ED), f32),
            pltpu.VMEM((NS, STRIPE), f32),
            pltpu.VMEM_SHARED((NPAD, ED), f32),
            pltpu.VMEM_SHARED((NS, NPAD), f32),
        ],
    )
    return kern(src, dst, a_src, a_dst, beta, ex)


def _finish_body(s_ref, d_ref, w_ref, o_ref):
    s = s_ref[0] + s_ref[1]
    d = d_ref[0] + d_ref[1]
    d = jnp.where(d == 0.0, 1.0, d)
    sw = s / d[:, None]
    o_ref[...] = jnp.dot(sw, w_ref[...].T, preferred_element_type=jnp.float32)


def _finish(s_parts, den_parts, W_e2n):
    blk = 1024
    grid = (NPAD // blk,)
    return pl.pallas_call(
        _finish_body,
        grid=grid,
        in_specs=[
            pl.BlockSpec((NC, blk, ED), lambda i: (0, i, 0)),
            pl.BlockSpec((NC, blk), lambda i: (0, i)),
            pl.BlockSpec((D, ED), lambda i: (0, 0)),
        ],
        out_specs=pl.BlockSpec((blk, D), lambda i: (i, 0)),
        out_shape=jax.ShapeDtypeStruct((NPAD, D), jnp.float32),
    )(s_parts, den_parts, W_e2n)


@jax.jit
def kernel(h, edge_index, e, W_att, W_edge, W_e2n):
    src = edge_index[0]
    dst = edge_index[1]
    e8 = e.reshape(E // 8, 8 * ED)

    eye8 = jnp.eye(8, dtype=jnp.float32)
    W1 = jnp.kron(eye8, W_edge.T)
    w3 = W_att[0, 2 * D:]
    W2 = jnp.kron(eye8, w3[:, None])

    a_src, a_dst = _node_alpha(h, W_att)
    ex8, beta8 = _edge_prep(e8, W1, W2)
    ex = ex8.reshape(E, ED)
    beta = beta8.reshape(E)

    s_parts, den_parts = _sc_agg(src, dst, a_src, a_dst, beta, ex)
    out = _finish(s_parts, den_parts, W_e2n)
    return out[:N]
